# baseline (speedup 1.0000x reference)
import jax
import jax.numpy as jnp
from jax import lax
from jax.experimental import pallas as pl
from jax.experimental.pallas import tpu as pltpu
from jax.experimental.pallas import tpu_sc as plsc

N = 10000
D = 128
E = 320000
NC = 2
NS = 16
LANES = 16

NPAD = 10240
DEG_PT = E // NS
RED_PT = NPAD // NS

CH = 88
RB = 20
NBLK = 12
NCHUNK = RB * NBLK
NBUF = 4
EDGES_PT = CH * NCHUNK
PE = EDGES_PT * NS
ACC_N = 10112
DUMP = N
ZROWS = ACC_N // NS


def _deg_body(ei_hbm, out_hbm, idx_v, hist_v, tmp_v, shist_s, sem):
    c = lax.axis_index("c")
    s = lax.axis_index("s")
    ones = jnp.ones((LANES,), jnp.float32)
    zeros = jnp.zeros((LANES,), jnp.float32)

    def zero_hist(i, _):
        hist_v[pl.ds(i * LANES, LANES)] = zeros
        return 0
    lax.fori_loop(0, NPAD // LANES, zero_hist, 0)

    pltpu.sync_copy(ei_hbm.at[c, s, 0], idx_v)

    def acc_body(i, _):
        idx = idx_v[pl.ds(i * LANES, LANES)]
        plsc.addupdate_scatter(hist_v, [idx], ones)
        return 0
    lax.fori_loop(0, DEG_PT // LANES, acc_body, 0)

    pltpu.sync_copy(hist_v, shist_s.at[s, 0])
    plsc.subcore_barrier()

    rbase = pl.multiple_of(s * RED_PT, 128)

    def zero_red(i, _):
        hist_v[pl.ds(i * LANES, LANES)] = zeros
        return 0
    lax.fori_loop(0, RED_PT // LANES, zero_red, 0)

    def red_body(t, _):
        pltpu.sync_copy(shist_s.at[t, 0, pl.ds(rbase, RED_PT)], tmp_v)

        def add_body(i, _):
            sl = pl.ds(i * LANES, LANES)
            hist_v[sl] = hist_v[sl] + tmp_v[sl]
            return 0
        lax.fori_loop(0, RED_PT // LANES, add_body, 0)
        return 0
    lax.fori_loop(0, NS, red_body, 0)

    def recip_body(i, _):
        sl = pl.ds(i * LANES, LANES)
        tmp_v[sl] = 1.0 / hist_v[sl]
        return 0
    lax.fori_loop(0, RED_PT // LANES, recip_body, 0)
    pltpu.sync_copy(tmp_v, out_hbm.at[c, 0, pl.ds(rbase, RED_PT)])


@jax.jit
def _deg_kernel(eidx):
    mesh = plsc.VectorSubcoreMesh(core_axis_name="c", subcore_axis_name="s")
    return pl.kernel(
        _deg_body,
        out_type=jax.ShapeDtypeStruct((NC, 1, NPAD), jnp.float32),
        mesh=mesh,
        compiler_params=pltpu.CompilerParams(needs_layout_passes=False),
        scratch_types=[
            pltpu.VMEM((DEG_PT,), jnp.int32),
            pltpu.VMEM((NPAD,), jnp.float32),
            pltpu.VMEM((RED_PT,), jnp.float32),
            pltpu.VMEM_SHARED((NS, 1, NPAD), jnp.float32),
            pltpu.SemaphoreType.DMA,
        ],
    )(eidx)


def _mp_body(xcat_hbm, gidx_hbm, sidx_hbm, ab_hbm,
             gbuf, sbuf, d0, d1, d2, d3, acc_s, sem0, sem1, sem2, sem3):
    c = lax.axis_index("c")
    s = lax.axis_index("s")
    zeros = jnp.zeros((LANES,), jnp.float32)
    dbuf = (d0, d1, d2, d3)
    gsem = (sem0, sem1, sem2, sem3)

    def zero_buf(i, _):
        r = i // (D // LANES)
        l = i % (D // LANES)
        d0[r, pl.ds(l * LANES, LANES)] = zeros
        return 0
    lax.fori_loop(0, CH * (D // LANES), zero_buf, 0)
    zbase = pl.multiple_of(s * ZROWS, 8)
    for r0 in range(0, ZROWS - CH + 1, CH):
        pltpu.sync_copy(d0, acc_s.at[pl.ds(zbase + r0, CH)])
    rem = ZROWS % CH
    if rem:
        pltpu.sync_copy(d0.at[pl.ds(0, rem)],
                        acc_s.at[pl.ds(zbase + (ZROWS - rem), rem)])
    plsc.subcore_barrier()

    def blk_body(bb, _):
        pltpu.sync_copy(gidx_hbm.at[c, s, pl.ds(bb * RB, RB)], gbuf)
        pltpu.sync_copy(sidx_hbm.at[c, s, pl.ds(bb * RB, RB)], sbuf)
        for t in range(NBUF):
            pltpu.async_copy(xcat_hbm.at[gbuf.at[t, 0]], dbuf[t], gsem[t])
        for j in range(RB):
            b = j % NBUF
            pltpu.make_async_copy(
                xcat_hbm.at[gbuf.at[j, 0]], dbuf[b], gsem[b]).wait()
            pltpu.sync_copy(dbuf[b], acc_s.at[sbuf.at[j, 0]], add=True)
            if j + NBUF < RB:
                pltpu.async_copy(
                    xcat_hbm.at[gbuf.at[j + NBUF, 0]], dbuf[b], gsem[b])
        return 0
    lax.fori_loop(0, NBLK, blk_body, 0)

    plsc.subcore_barrier()

    obase = pl.multiple_of(s * ZROWS, 8)
    pltpu.sync_copy(acc_s.at[pl.ds(obase, ZROWS)],
                    ab_hbm.at[c, pl.ds(obase, ZROWS)])


@jax.jit
def _mp_kernel(xcat, gidx, sidx):
    mesh = plsc.VectorSubcoreMesh(core_axis_name="c", subcore_axis_name="s")
    return pl.kernel(
        _mp_body,
        out_type=jax.ShapeDtypeStruct((NC, ACC_N, D), jnp.float32),
        mesh=mesh,
        scratch_types=[
            pltpu.VMEM((RB, 1, CH), jnp.int32),
            pltpu.VMEM((RB, 1, CH), jnp.int32),
            pltpu.VMEM((CH, D), jnp.float32),
            pltpu.VMEM((CH, D), jnp.float32),
            pltpu.VMEM((CH, D), jnp.float32),
            pltpu.VMEM((CH, D), jnp.float32),
            pltpu.VMEM_SHARED((ACC_N, D), jnp.float32),
            pltpu.SemaphoreType.DMA,
            pltpu.SemaphoreType.DMA,
            pltpu.SemaphoreType.DMA,
            pltpu.SemaphoreType.DMA,
        ],
    )(xcat, gidx, sidx)


BN = 1000


def _scale_body(x_ref, ro_ref, ri_ref, o_ref):
    x = x_ref[...]
    o_ref[0] = x * ro_ref[...]
    o_ref[1] = x * ri_ref[...]


@jax.jit
def _scale_kernel(x, ro, ri):
    return pl.pallas_call(
        _scale_body,
        grid=(N // BN,),
        in_specs=[
            pl.BlockSpec((BN, D), lambda i: (i, 0)),
            pl.BlockSpec((BN, 1), lambda i: (i, 0)),
            pl.BlockSpec((BN, 1), lambda i: (i, 0)),
        ],
        out_specs=pl.BlockSpec((NC, BN, D), lambda i: (0, i, 0)),
        out_shape=jax.ShapeDtypeStruct((NC, N, D), jnp.float32),
    )(x, ro, ri)


def _gate_body(x_ref, a_ref, b_ref, wc_ref, bc_ref, o_ref):
    m = jnp.concatenate([x_ref[...], a_ref[...], b_ref[...]], axis=1)
    p = jnp.dot(m, wc_ref[...], preferred_element_type=jnp.float32)
    p = p + bc_ref[...]
    z = jax.nn.sigmoid(p[:, :D])
    h = jnp.tanh(p[:, D:])
    o_ref[...] = (1.0 - z) * h


@jax.jit
def _gate_kernel(x, a, b, wc, bc):
    return pl.pallas_call(
        _gate_body,
        grid=(N // BN,),
        in_specs=[
            pl.BlockSpec((BN, D), lambda i: (i, 0)),
            pl.BlockSpec((BN, D), lambda i: (i, 0)),
            pl.BlockSpec((BN, D), lambda i: (i, 0)),
            pl.BlockSpec((3 * D, 2 * D), lambda i: (0, 0)),
            pl.BlockSpec((1, 2 * D), lambda i: (0, 0)),
        ],
        out_specs=pl.BlockSpec((BN, D), lambda i: (i, 0)),
        out_shape=jax.ShapeDtypeStruct((N, D), jnp.float32),
    )(x, a, b, wc, bc)


def kernel(x, edge_index, Wz, bz, Wr, br, Wh, bh):
    row = edge_index[0]
    col = edge_index[1]
    L = Wz.shape[0]

    pad = PE - E
    g0 = jnp.concatenate([row, jnp.zeros((pad,), jnp.int32)])
    g1 = jnp.concatenate([col, jnp.zeros((pad,), jnp.int32)]) + N
    s0 = jnp.concatenate([col, jnp.full((pad,), DUMP, jnp.int32)])
    s1 = jnp.concatenate([row, jnp.full((pad,), DUMP, jnp.int32)])
    gidx = jnp.stack([g0, g1]).reshape(NC, NS, NCHUNK, 1, CH)
    sidx = jnp.stack([s0, s1]).reshape(NC, NS, NCHUNK, 1, CH)
    eidx = edge_index.reshape(NC, NS, 1, DEG_PT)

    recip = _deg_kernel(eidx)
    ro = recip[0, 0, :N].reshape(N, 1)
    ri = recip[1, 0, :N].reshape(N, 1)

    def wcat(l):
        wz0 = Wz[l, 0, 0, :D] + Wz[l, 1, 0, :D]
        wh0 = Wh[l, 0, 0, :D] + Wh[l, 1, 0, :D]
        return jnp.concatenate([
            jnp.concatenate([wz0, wh0], axis=1),
            jnp.concatenate([Wz[l, 0, 1, :D], Wh[l, 0, 1, :D]], axis=1),
            jnp.concatenate([Wz[l, 1, 1, :D], Wh[l, 1, 1, :D]], axis=1),
        ], axis=0)

    X = x
    for l in range(L):
        xcat = _scale_kernel(X, ro, ri).reshape(NC * N, D)
        ab = _mp_kernel(xcat, gidx, sidx)
        bc = jnp.concatenate([bz[l], bh[l]]).reshape(1, 2 * D)
        X = _gate_kernel(X, ab[0, :N], ab[1, :N], wcat(l), bc)
    return X


# speedup vs baseline: 3.0172x; 3.0172x over previous
import jax
import jax.numpy as jnp
from jax import lax
from jax.experimental import pallas as pl
from jax.experimental.pallas import tpu as pltpu
from jax.experimental.pallas import tpu_sc as plsc

N = 10000
D = 128
E = 320000
NC = 2
NS = 16
LANES = 16

NPAD = 10240
DEG_PT = E // NS
RED_PT = NPAD // NS

CH = 120
RB = 12
NBLK = 14
NCHUNK = RB * NBLK
NBUF = 3
EDGES_PT = CH * NCHUNK
PE = EDGES_PT * NS
ACC_N = 10112
DUMP = N
ZROWS = ACC_N // NS


def _deg_body(ei_hbm, out_hbm, idx_v, hist_v, tmp_v, shist_s, sem):
    c = lax.axis_index("c")
    s = lax.axis_index("s")
    ones = jnp.ones((LANES,), jnp.float32)
    zeros = jnp.zeros((LANES,), jnp.float32)

    def zero_hist(i, _):
        hist_v[pl.ds(i * LANES, LANES)] = zeros
        return 0
    lax.fori_loop(0, NPAD // LANES, zero_hist, 0)

    pltpu.sync_copy(ei_hbm.at[c, s, 0], idx_v)

    def acc_body(i, _):
        idx = idx_v[pl.ds(i * LANES, LANES)]
        plsc.addupdate_scatter(hist_v, [idx], ones)
        return 0
    lax.fori_loop(0, DEG_PT // LANES, acc_body, 0)

    pltpu.sync_copy(hist_v, shist_s.at[s, 0])
    plsc.subcore_barrier()

    rbase = pl.multiple_of(s * RED_PT, 128)

    def zero_red(i, _):
        hist_v[pl.ds(i * LANES, LANES)] = zeros
        return 0
    lax.fori_loop(0, RED_PT // LANES, zero_red, 0)

    def red_body(t, _):
        pltpu.sync_copy(shist_s.at[t, 0, pl.ds(rbase, RED_PT)], tmp_v)

        def add_body(i, _):
            sl = pl.ds(i * LANES, LANES)
            hist_v[sl] = hist_v[sl] + tmp_v[sl]
            return 0
        lax.fori_loop(0, RED_PT // LANES, add_body, 0)
        return 0
    lax.fori_loop(0, NS, red_body, 0)

    def recip_body(i, _):
        sl = pl.ds(i * LANES, LANES)
        tmp_v[sl] = 1.0 / hist_v[sl]
        return 0
    lax.fori_loop(0, RED_PT // LANES, recip_body, 0)
    pltpu.sync_copy(tmp_v, out_hbm.at[c, 0, pl.ds(rbase, RED_PT)])


@jax.jit
def _deg_kernel(eidx):
    mesh = plsc.VectorSubcoreMesh(core_axis_name="c", subcore_axis_name="s")
    return pl.kernel(
        _deg_body,
        out_type=jax.ShapeDtypeStruct((NC, 1, NPAD), jnp.float32),
        mesh=mesh,
        compiler_params=pltpu.CompilerParams(needs_layout_passes=False),
        scratch_types=[
            pltpu.VMEM((DEG_PT,), jnp.int32),
            pltpu.VMEM((NPAD,), jnp.float32),
            pltpu.VMEM((RED_PT,), jnp.float32),
            pltpu.VMEM_SHARED((NS, 1, NPAD), jnp.float32),
            pltpu.SemaphoreType.DMA,
        ],
    )(eidx)


def _mp_body(xcat_hbm, gidx_hbm, sidx_hbm, ab_hbm,
             gbuf, sbuf, d0, d1, d2, acc_s, sem0, sem1, sem2):
    c = lax.axis_index("c")
    s = lax.axis_index("s")
    zeros = jnp.zeros((LANES,), jnp.float32)
    dbuf = (d0, d1, d2)
    gsem = (sem0, sem1, sem2)

    def zero_buf(i, _):
        r = i // (D // LANES)
        l = i % (D // LANES)
        d0[r, pl.ds(l * LANES, LANES)] = zeros
        return 0
    lax.fori_loop(0, CH * (D // LANES), zero_buf, 0)
    zbase = pl.multiple_of(s * ZROWS, 8)
    for r0 in range(0, ZROWS - CH + 1, CH):
        pltpu.sync_copy(d0, acc_s.at[pl.ds(zbase + r0, CH)])
    rem = ZROWS % CH
    if rem:
        pltpu.sync_copy(d0.at[pl.ds(0, rem)],
                        acc_s.at[pl.ds(zbase + (ZROWS - rem), rem)])
    plsc.subcore_barrier()

    def blk_body(bb, _):
        pltpu.sync_copy(gidx_hbm.at[c, s, pl.ds(bb * RB, RB)], gbuf)
        pltpu.sync_copy(sidx_hbm.at[c, s, pl.ds(bb * RB, RB)], sbuf)
        for t in range(NBUF):
            pltpu.async_copy(xcat_hbm.at[gbuf.at[t, 0]], dbuf[t], gsem[t])
        for j in range(RB):
            b = j % NBUF
            pltpu.make_async_copy(
                xcat_hbm.at[gbuf.at[j, 0]], dbuf[b], gsem[b]).wait()
            pltpu.sync_copy(dbuf[b], acc_s.at[sbuf.at[j, 0]], add=True)
            if j + NBUF < RB:
                pltpu.async_copy(
                    xcat_hbm.at[gbuf.at[j + NBUF, 0]], dbuf[b], gsem[b])
        return 0
    lax.fori_loop(0, NBLK, blk_body, 0)

    plsc.subcore_barrier()

    obase = pl.multiple_of(s * ZROWS, 8)
    pltpu.sync_copy(acc_s.at[pl.ds(obase, ZROWS)],
                    ab_hbm.at[c, pl.ds(obase, ZROWS)])


@jax.jit
def _mp_kernel(xcat, gidx, sidx):
    mesh = plsc.VectorSubcoreMesh(core_axis_name="c", subcore_axis_name="s")
    return pl.kernel(
        _mp_body,
        out_type=jax.ShapeDtypeStruct((NC, ACC_N, D), jnp.float32),
        mesh=mesh,
        scratch_types=[
            pltpu.VMEM((RB, 1, CH), jnp.int32),
            pltpu.VMEM((RB, 1, CH), jnp.int32),
            pltpu.VMEM((CH, D), jnp.float32),
            pltpu.VMEM((CH, D), jnp.float32),
            pltpu.VMEM((CH, D), jnp.float32),
            pltpu.VMEM_SHARED((ACC_N, D), jnp.float32),
            pltpu.SemaphoreType.DMA,
            pltpu.SemaphoreType.DMA,
            pltpu.SemaphoreType.DMA,
        ],
    )(xcat, gidx, sidx)


BN = 1000


def _scale_body(x_ref, ro_ref, ri_ref, o_ref):
    x = x_ref[...]
    o_ref[0] = x * ro_ref[...]
    o_ref[1] = x * ri_ref[...]


@jax.jit
def _scale_kernel(x, ro, ri):
    return pl.pallas_call(
        _scale_body,
        grid=(N // BN,),
        in_specs=[
            pl.BlockSpec((BN, D), lambda i: (i, 0)),
            pl.BlockSpec((BN, 1), lambda i: (i, 0)),
            pl.BlockSpec((BN, 1), lambda i: (i, 0)),
        ],
        out_specs=pl.BlockSpec((NC, BN, D), lambda i: (0, i, 0)),
        out_shape=jax.ShapeDtypeStruct((NC, N, D), jnp.float32),
    )(x, ro, ri)


def _gate_body(x_ref, a_ref, b_ref, wc_ref, bc_ref, o_ref):
    m = jnp.concatenate([x_ref[...], a_ref[...], b_ref[...]], axis=1)
    p = jnp.dot(m, wc_ref[...], preferred_element_type=jnp.float32)
    p = p + bc_ref[...]
    z = jax.nn.sigmoid(p[:, :D])
    h = jnp.tanh(p[:, D:])
    o_ref[...] = (1.0 - z) * h


@jax.jit
def _gate_kernel(x, a, b, wc, bc):
    return pl.pallas_call(
        _gate_body,
        grid=(N // BN,),
        in_specs=[
            pl.BlockSpec((BN, D), lambda i: (i, 0)),
            pl.BlockSpec((BN, D), lambda i: (i, 0)),
            pl.BlockSpec((BN, D), lambda i: (i, 0)),
            pl.BlockSpec((3 * D, 2 * D), lambda i: (0, 0)),
            pl.BlockSpec((1, 2 * D), lambda i: (0, 0)),
        ],
        out_specs=pl.BlockSpec((BN, D), lambda i: (i, 0)),
        out_shape=jax.ShapeDtypeStruct((N, D), jnp.float32),
    )(x, a, b, wc, bc)


def kernel(x, edge_index, Wz, bz, Wr, br, Wh, bh):
    row = edge_index[0]
    col = edge_index[1]
    L = Wz.shape[0]

    pad = PE - E
    g0 = jnp.concatenate([row, jnp.zeros((pad,), jnp.int32)])
    g1 = jnp.concatenate([col, jnp.zeros((pad,), jnp.int32)]) + N
    s0 = jnp.concatenate([col, jnp.full((pad,), DUMP, jnp.int32)])
    s1 = jnp.concatenate([row, jnp.full((pad,), DUMP, jnp.int32)])
    gidx = jnp.stack([g0, g1]).reshape(NC, NS, NCHUNK, 1, CH)
    sidx = jnp.stack([s0, s1]).reshape(NC, NS, NCHUNK, 1, CH)
    eidx = edge_index.reshape(NC, NS, 1, DEG_PT)

    recip = _deg_kernel(eidx)
    ro = recip[0, 0, :N].reshape(N, 1)
    ri = recip[1, 0, :N].reshape(N, 1)

    def wcat(l):
        wz0 = Wz[l, 0, 0, :D] + Wz[l, 1, 0, :D]
        wh0 = Wh[l, 0, 0, :D] + Wh[l, 1, 0, :D]
        return jnp.concatenate([
            jnp.concatenate([wz0, wh0], axis=1),
            jnp.concatenate([Wz[l, 0, 1, :D], Wh[l, 0, 1, :D]], axis=1),
            jnp.concatenate([Wz[l, 1, 1, :D], Wh[l, 1, 1, :D]], axis=1),
        ], axis=0)

    X = x
    for l in range(L):
        xcat = _scale_kernel(X, ro, ri).reshape(NC * N, D)
        ab = _mp_kernel(xcat, gidx, sidx)
        bc = jnp.concatenate([bz[l], bh[l]]).reshape(1, 2 * D)
        X = _gate_kernel(X, ab[0, :N], ab[1, :N], wcat(l), bc)
    return X


# speedup vs baseline: 3.3415x; 1.1075x over previous
import jax
import jax.numpy as jnp
from jax import lax
from jax.experimental import pallas as pl
from jax.experimental.pallas import tpu as pltpu
from jax.experimental.pallas import tpu_sc as plsc

N = 10000
D = 128
E = 320000
NC = 2
NS = 16
LANES = 16

NPAD = 10240
DEG_PT = E // NS
RED_PT = NPAD // NS

CH = 120
RB = 12
NBLK = 14
NCHUNK = RB * NBLK
NBUF = 3
EDGES_PT = CH * NCHUNK
PE = EDGES_PT * NS
ACC_N = 10112
DUMP = N
ZROWS = ACC_N // NS


def _deg_body(ei_hbm, out_hbm, idx_v, hist_v, tmp_v, shist_s, sem):
    c = lax.axis_index("c")
    s = lax.axis_index("s")
    ones = jnp.ones((LANES,), jnp.float32)
    zeros = jnp.zeros((LANES,), jnp.float32)

    def zero_hist(i, _):
        hist_v[pl.ds(i * LANES, LANES)] = zeros
        return 0
    lax.fori_loop(0, NPAD // LANES, zero_hist, 0)

    pltpu.sync_copy(ei_hbm.at[c, s, 0], idx_v)

    def acc_body(i, _):
        idx = idx_v[pl.ds(i * LANES, LANES)]
        plsc.addupdate_scatter(hist_v, [idx], ones)
        return 0
    lax.fori_loop(0, DEG_PT // LANES, acc_body, 0)

    pltpu.sync_copy(hist_v, shist_s.at[s, 0])
    plsc.subcore_barrier()

    rbase = pl.multiple_of(s * RED_PT, 128)

    def zero_red(i, _):
        hist_v[pl.ds(i * LANES, LANES)] = zeros
        return 0
    lax.fori_loop(0, RED_PT // LANES, zero_red, 0)

    def red_body(t, _):
        pltpu.sync_copy(shist_s.at[t, 0, pl.ds(rbase, RED_PT)], tmp_v)

        def add_body(i, _):
            sl = pl.ds(i * LANES, LANES)
            hist_v[sl] = hist_v[sl] + tmp_v[sl]
            return 0
        lax.fori_loop(0, RED_PT // LANES, add_body, 0)
        return 0
    lax.fori_loop(0, NS, red_body, 0)

    def recip_body(i, _):
        sl = pl.ds(i * LANES, LANES)
        tmp_v[sl] = 1.0 / hist_v[sl]
        return 0
    lax.fori_loop(0, RED_PT // LANES, recip_body, 0)
    pltpu.sync_copy(tmp_v, out_hbm.at[c, 0, pl.ds(rbase, RED_PT)])


@jax.jit
def _deg_kernel(eidx):
    mesh = plsc.VectorSubcoreMesh(core_axis_name="c", subcore_axis_name="s")
    return pl.kernel(
        _deg_body,
        out_type=jax.ShapeDtypeStruct((NC, 1, NPAD), jnp.float32),
        mesh=mesh,
        compiler_params=pltpu.CompilerParams(needs_layout_passes=False),
        scratch_types=[
            pltpu.VMEM((DEG_PT,), jnp.int32),
            pltpu.VMEM((NPAD,), jnp.float32),
            pltpu.VMEM((RED_PT,), jnp.float32),
            pltpu.VMEM_SHARED((NS, 1, NPAD), jnp.float32),
            pltpu.SemaphoreType.DMA,
        ],
    )(eidx)


def _mp_body(xcat_hbm, gidx_hbm, sidx_hbm, ab_hbm,
             gbuf, sbuf, d0, d1, d2, acc_s, sem0, sem1, sem2):
    c = lax.axis_index("c")
    s = lax.axis_index("s")
    zeros = jnp.zeros((LANES,), jnp.float32)
    dbuf = (d0, d1, d2)
    gsem = (sem0, sem1, sem2)

    def zero_buf(i, _):
        r = i // (D // LANES)
        l = i % (D // LANES)
        d0[r, pl.ds(l * LANES, LANES)] = zeros
        return 0
    lax.fori_loop(0, CH * (D // LANES), zero_buf, 0)
    zbase = pl.multiple_of(s * ZROWS, 8)
    for r0 in range(0, ZROWS - CH + 1, CH):
        pltpu.sync_copy(d0, acc_s.at[pl.ds(zbase + r0, CH)])
    rem = ZROWS % CH
    if rem:
        pltpu.sync_copy(d0.at[pl.ds(0, rem)],
                        acc_s.at[pl.ds(zbase + (ZROWS - rem), rem)])
    plsc.subcore_barrier()

    def blk_body(bb, _):
        pltpu.sync_copy(gidx_hbm.at[c, s, pl.ds(bb * RB, RB)], gbuf)
        pltpu.sync_copy(sidx_hbm.at[c, s, pl.ds(bb * RB, RB)], sbuf)
        for t in range(NBUF):
            pltpu.async_copy(xcat_hbm.at[gbuf.at[t, 0]], dbuf[t], gsem[t])
        for j in range(RB):
            b = j % NBUF
            pltpu.make_async_copy(
                xcat_hbm.at[gbuf.at[j, 0]], dbuf[b], gsem[b]).wait()
            pltpu.sync_copy(dbuf[b], acc_s.at[sbuf.at[j, 0]], add=True)
            if j + NBUF < RB:
                pltpu.async_copy(
                    xcat_hbm.at[gbuf.at[j + NBUF, 0]], dbuf[b], gsem[b])
        return 0
    lax.fori_loop(0, NBLK, blk_body, 0)

    plsc.subcore_barrier()

    obase = pl.multiple_of(s * ZROWS, 8)
    pltpu.sync_copy(acc_s.at[pl.ds(obase, ZROWS)],
                    ab_hbm.at[c, pl.ds(obase, ZROWS)])


@jax.jit
def _mp_kernel(xcat, gidx, sidx):
    mesh = plsc.VectorSubcoreMesh(core_axis_name="c", subcore_axis_name="s")
    return pl.kernel(
        _mp_body,
        out_type=jax.ShapeDtypeStruct((NC, ACC_N, D), jnp.float32),
        mesh=mesh,
        scratch_types=[
            pltpu.VMEM((RB, 1, CH), jnp.int32),
            pltpu.VMEM((RB, 1, CH), jnp.int32),
            pltpu.VMEM((CH, D), jnp.float32),
            pltpu.VMEM((CH, D), jnp.float32),
            pltpu.VMEM((CH, D), jnp.float32),
            pltpu.VMEM_SHARED((ACC_N, D), jnp.float32),
            pltpu.SemaphoreType.DMA,
            pltpu.SemaphoreType.DMA,
            pltpu.SemaphoreType.DMA,
        ],
    )(xcat, gidx, sidx)


BN = 1000


def _scale_body(x_ref, ro_ref, ri_ref, o_ref):
    x = x_ref[...]
    o_ref[0] = x * ro_ref[...]
    o_ref[1] = x * ri_ref[...]


@jax.jit
def _scale_kernel(x, ro, ri):
    return pl.pallas_call(
        _scale_body,
        grid=(N // BN,),
        in_specs=[
            pl.BlockSpec((BN, D), lambda i: (i, 0)),
            pl.BlockSpec((BN, 1), lambda i: (i, 0)),
            pl.BlockSpec((BN, 1), lambda i: (i, 0)),
        ],
        out_specs=pl.BlockSpec((NC, BN, D), lambda i: (0, i, 0)),
        out_shape=jax.ShapeDtypeStruct((NC, N, D), jnp.float32),
    )(x, ro, ri)


def _gate_body(x_ref, a_ref, b_ref, wc_ref, bc_ref, o_ref):
    m = jnp.concatenate([x_ref[...], a_ref[...], b_ref[...]], axis=1)
    p = jnp.dot(m, wc_ref[...], preferred_element_type=jnp.float32)
    p = p + bc_ref[...]
    z = jax.nn.sigmoid(p[:, :D])
    h = jnp.tanh(p[:, D:])
    o_ref[...] = (1.0 - z) * h


def _gate_scale_body(x_ref, a_ref, b_ref, wc_ref, bc_ref, ro_ref, ri_ref,
                     o_ref, oc_ref):
    m = jnp.concatenate([x_ref[...], a_ref[...], b_ref[...]], axis=1)
    p = jnp.dot(m, wc_ref[...], preferred_element_type=jnp.float32)
    p = p + bc_ref[...]
    z = jax.nn.sigmoid(p[:, :D])
    h = jnp.tanh(p[:, D:])
    o = (1.0 - z) * h
    o_ref[...] = o
    oc_ref[0] = o * ro_ref[...]
    oc_ref[1] = o * ri_ref[...]


@jax.jit
def _gate_scale_kernel(x, a, b, wc, bc, ro, ri):
    return pl.pallas_call(
        _gate_scale_body,
        grid=(N // BN,),
        in_specs=[
            pl.BlockSpec((BN, D), lambda i: (i, 0)),
            pl.BlockSpec((BN, D), lambda i: (i, 0)),
            pl.BlockSpec((BN, D), lambda i: (i, 0)),
            pl.BlockSpec((3 * D, 2 * D), lambda i: (0, 0)),
            pl.BlockSpec((1, 2 * D), lambda i: (0, 0)),
            pl.BlockSpec((BN, 1), lambda i: (i, 0)),
            pl.BlockSpec((BN, 1), lambda i: (i, 0)),
        ],
        out_specs=[
            pl.BlockSpec((BN, D), lambda i: (i, 0)),
            pl.BlockSpec((NC, BN, D), lambda i: (0, i, 0)),
        ],
        out_shape=[
            jax.ShapeDtypeStruct((N, D), jnp.float32),
            jax.ShapeDtypeStruct((NC, N, D), jnp.float32),
        ],
    )(x, a, b, wc, bc, ro, ri)


@jax.jit
def _gate_kernel(x, a, b, wc, bc):
    return pl.pallas_call(
        _gate_body,
        grid=(N // BN,),
        in_specs=[
            pl.BlockSpec((BN, D), lambda i: (i, 0)),
            pl.BlockSpec((BN, D), lambda i: (i, 0)),
            pl.BlockSpec((BN, D), lambda i: (i, 0)),
            pl.BlockSpec((3 * D, 2 * D), lambda i: (0, 0)),
            pl.BlockSpec((1, 2 * D), lambda i: (0, 0)),
        ],
        out_specs=pl.BlockSpec((BN, D), lambda i: (i, 0)),
        out_shape=jax.ShapeDtypeStruct((N, D), jnp.float32),
    )(x, a, b, wc, bc)


def kernel(x, edge_index, Wz, bz, Wr, br, Wh, bh):
    row = edge_index[0]
    col = edge_index[1]
    L = Wz.shape[0]

    pad = PE - E
    g0 = jnp.concatenate([row, jnp.zeros((pad,), jnp.int32)])
    g1 = jnp.concatenate([col, jnp.zeros((pad,), jnp.int32)]) + N
    s0 = jnp.concatenate([col, jnp.full((pad,), DUMP, jnp.int32)])
    s1 = jnp.concatenate([row, jnp.full((pad,), DUMP, jnp.int32)])
    gidx = jnp.stack([g0, g1]).reshape(NC, NS, NCHUNK, 1, CH)
    sidx = jnp.stack([s0, s1]).reshape(NC, NS, NCHUNK, 1, CH)
    eidx = edge_index.reshape(NC, NS, 1, DEG_PT)

    recip = _deg_kernel(eidx)
    ro = recip[0, 0, :N].reshape(N, 1)
    ri = recip[1, 0, :N].reshape(N, 1)

    def wcat(l):
        wz0 = Wz[l, 0, 0, :D] + Wz[l, 1, 0, :D]
        wh0 = Wh[l, 0, 0, :D] + Wh[l, 1, 0, :D]
        return jnp.concatenate([
            jnp.concatenate([wz0, wh0], axis=1),
            jnp.concatenate([Wz[l, 0, 1, :D], Wh[l, 0, 1, :D]], axis=1),
            jnp.concatenate([Wz[l, 1, 1, :D], Wh[l, 1, 1, :D]], axis=1),
        ], axis=0)

    X = x
    xcat = _scale_kernel(X, ro, ri).reshape(NC * N, D)
    for l in range(L):
        ab = _mp_kernel(xcat, gidx, sidx)
        bc = jnp.concatenate([bz[l], bh[l]]).reshape(1, 2 * D)
        if l + 1 < L:
            X, xc = _gate_scale_kernel(X, ab[0, :N], ab[1, :N], wcat(l), bc,
                                       ro, ri)
            xcat = xc.reshape(NC * N, D)
        else:
            X = _gate_kernel(X, ab[0, :N], ab[1, :N], wcat(l), bc)
    return X
